# baseline (speedup 1.0000x reference)
import jax
import jax.numpy as jnp
from jax import lax
from jax.experimental import pallas as pl
from jax.experimental.pallas import tpu as pltpu
from jax.experimental.pallas import tpu_sc as plsc

_EMB = 32
_WIDE = 128
_BATCH = 16384
_NUM_ENT = 1000000
_EBLK = 512
_NUM_CORES = 2
_NUM_SUBCORES = 16
_LANES = 16
_NW = _NUM_CORES * _NUM_SUBCORES
_BPW = _BATCH // _NW
_ROUND = 128
_NROUNDS = _BPW // _ROUND
_RCHUNKS = _ROUND // _LANES


def _tc_repack_body(src_ref, dst_ref):
  for k in range(4):
    dst_ref[:, k * _EMB:(k + 1) * _EMB] = jnp.transpose(
        src_ref[:, k * _WIDE:(k + 1) * _WIDE], (1, 0))


_tc_repack = pl.pallas_call(
    _tc_repack_body,
    grid=(pl.cdiv(_NUM_ENT, _EBLK),),
    in_specs=[pl.BlockSpec((_EMB, _EBLK), lambda b: (0, b))],
    out_specs=pl.BlockSpec((_WIDE, _WIDE), lambda b: (b, 0)),
    out_shape=jax.ShapeDtypeStruct((_NUM_ENT // 4, _WIDE), jnp.float32),
)


def _tec_body(h_idx, r_idx, t_idx, time_idx, ent, rel, tim, out,
              hi_v, ri_v, ti_v, mi_v, hr_v, rr_v, tr_v, mr_v,
              h_v, r_v, t_v, m_v, o_v, sem):
  wid = lax.axis_index("s") * _NUM_CORES + lax.axis_index("c")
  base = wid * _BPW

  pltpu.sync_copy(h_idx.at[pl.ds(base, _BPW)], hi_v)
  pltpu.sync_copy(r_idx.at[pl.ds(base, _BPW)], ri_v)
  pltpu.sync_copy(t_idx.at[pl.ds(base, _BPW)], ti_v)
  pltpu.sync_copy(time_idx.at[pl.ds(base, _BPW)], mi_v)

  last_lane = lax.iota(jnp.int32, _LANES) == (_LANES - 1)

  def ent_row(i):
    return lax.bitwise_or(
        lax.shift_left(lax.shift_right_logical(i, 9), 7),
        lax.bitwise_and(i, 127))

  def do_round(rnd, carry):
    rbase = rnd * _ROUND

    def prep(c, carry2):
      s = rbase + c * _LANES
      hr_v[pl.ds(c * _LANES, _LANES)] = ent_row(hi_v[pl.ds(s, _LANES)])
      tr_v[pl.ds(c * _LANES, _LANES)] = ent_row(ti_v[pl.ds(s, _LANES)])
      rr_v[pl.ds(c * _LANES, _LANES)] = lax.shift_right_logical(
          ri_v[pl.ds(s, _LANES)], 2)
      mr_v[pl.ds(c * _LANES, _LANES)] = lax.shift_right_logical(
          mi_v[pl.ds(s, _LANES)], 2)
      return carry2

    lax.fori_loop(0, _RCHUNKS, prep, 0)

    c1 = pltpu.async_copy(ent.at[hr_v], h_v, sem)
    c2 = pltpu.async_copy(ent.at[tr_v], t_v, sem)
    c3 = pltpu.async_copy(rel.at[rr_v], r_v, sem)
    c4 = pltpu.async_copy(tim.at[mr_v], m_v, sem)
    c1.wait()
    c2.wait()
    c3.wait()
    c4.wait()

    def chunk(c, carry2):
      s = rbase + c * _LANES
      oh = (lax.shift_right_logical(hi_v[pl.ds(s, _LANES)], 7) & 3) << 5
      ot = (lax.shift_right_logical(ti_v[pl.ds(s, _LANES)], 7) & 3) << 5
      orr = (ri_v[pl.ds(s, _LANES)] & 3) << 5
      om = (mi_v[pl.ds(s, _LANES)] & 3) << 5
      for j in range(_LANES):
        e = c * _LANES + j
        offh = oh[j]
        offt = ot[j]
        offr = orr[j]
        offm = om[j]
        h0 = h_v[e, pl.ds(offh, _LANES)]
        h1 = h_v[e, pl.ds(offh + _LANES, _LANES)]
        t0 = t_v[e, pl.ds(offt, _LANES)]
        t1 = t_v[e, pl.ds(offt + _LANES, _LANES)]
        r0 = r_v[e, pl.ds(offr, _LANES)]
        r1 = r_v[e, pl.ds(offr + _LANES, _LANES)]
        m0 = m_v[e, pl.ds(offm, _LANES)]
        m1 = m_v[e, pl.ds(offm + _LANES, _LANES)]
        sv = (0.0 - jnp.abs(h0 + r0 + m0 - t0)) - jnp.abs(h1 + r1 + m1 - t1)
        cs = plsc.cumsum(sv)
        plsc.store_scatter(
            o_v, [jnp.full((_LANES,), rbase + e, jnp.int32)], cs,
            mask=last_lane)
      return carry2

    lax.fori_loop(0, _RCHUNKS, chunk, 0)
    return carry

  lax.fori_loop(0, _NROUNDS, do_round, 0)
  pltpu.sync_copy(o_v, out.at[pl.ds(base, _BPW)])


_mesh = plsc.VectorSubcoreMesh(
    core_axis_name="c", subcore_axis_name="s",
    num_cores=_NUM_CORES, num_subcores=_NUM_SUBCORES)

_sc_call = pl.kernel(
    _tec_body,
    out_type=jax.ShapeDtypeStruct((_BATCH,), jnp.float32),
    mesh=_mesh,
    compiler_params=pltpu.CompilerParams(needs_layout_passes=False),
    scratch_types=[
        pltpu.VMEM((_BPW,), jnp.int32),
        pltpu.VMEM((_BPW,), jnp.int32),
        pltpu.VMEM((_BPW,), jnp.int32),
        pltpu.VMEM((_BPW,), jnp.int32),
        pltpu.VMEM((_ROUND,), jnp.int32),
        pltpu.VMEM((_ROUND,), jnp.int32),
        pltpu.VMEM((_ROUND,), jnp.int32),
        pltpu.VMEM((_ROUND,), jnp.int32),
        pltpu.VMEM((_ROUND, _WIDE), jnp.float32),
        pltpu.VMEM((_ROUND, _WIDE), jnp.float32),
        pltpu.VMEM((_ROUND, _WIDE), jnp.float32),
        pltpu.VMEM((_ROUND, _WIDE), jnp.float32),
        pltpu.VMEM((_BPW,), jnp.float32),
        pltpu.SemaphoreType.DMA,
    ],
)


@jax.jit
def kernel(h_idx, r_idx, t_idx, time_idx, entity_emb, relation_emb, time_emb):
  ent128 = _tc_repack(entity_emb.T)
  rel128 = relation_emb.reshape(250, _WIDE)
  tim128 = jnp.pad(time_emb.reshape(-1), (0, 96)).reshape(92, _WIDE)
  return _sc_call(
      h_idx.astype(jnp.int32), r_idx.astype(jnp.int32),
      t_idx.astype(jnp.int32), time_idx.astype(jnp.int32),
      ent128, rel128, tim128)


# speedup vs baseline: 3.1738x; 3.1738x over previous
import jax
import jax.numpy as jnp
from jax import lax
from jax.experimental import pallas as pl
from jax.experimental.pallas import tpu as pltpu
from jax.experimental.pallas import tpu_sc as plsc

_EMB = 32
_WIDE = 128
_BATCH = 16384
_NUM_CORES = 2
_NUM_SUBCORES = 16
_LANES = 16
_NW = _NUM_CORES * _NUM_SUBCORES
_BPW = _BATCH // _NW
_ROUND = 32
_NROUNDS = _BPW // _ROUND
_RCHUNKS = _ROUND // _LANES


def _tec_body(h_idx, r_idx, t_idx, time_idx, ent, rel, tim, out,
              hi_v, ri_v, ti_v, mi_v, rr_v, mr_v,
              h_v, t_v, r_v, m_v, o_v, sem):
  wid = lax.axis_index("s") * _NUM_CORES + lax.axis_index("c")
  base = wid * _BPW

  pltpu.sync_copy(h_idx.at[pl.ds(base, _BPW)], hi_v)
  pltpu.sync_copy(r_idx.at[pl.ds(base, _BPW)], ri_v)
  pltpu.sync_copy(t_idx.at[pl.ds(base, _BPW)], ti_v)
  pltpu.sync_copy(time_idx.at[pl.ds(base, _BPW)], mi_v)

  last_lane = lax.iota(jnp.int32, _LANES) == (_LANES - 1)

  def do_round(rnd, carry):
    rbase = rnd * _ROUND

    def prep(c, carry2):
      s = rbase + c * _LANES
      rr_v[pl.ds(c * _LANES, _LANES)] = lax.shift_right_logical(
          ri_v[pl.ds(s, _LANES)], 2)
      mr_v[pl.ds(c * _LANES, _LANES)] = lax.shift_right_logical(
          mi_v[pl.ds(s, _LANES)], 2)
      return carry2

    lax.fori_loop(0, _RCHUNKS, prep, 0)

    c3 = pltpu.async_copy(rel.at[rr_v], r_v, sem)
    c4 = pltpu.async_copy(tim.at[mr_v], m_v, sem)

    cps = []
    for c in range(_RCHUNKS):
      hv = hi_v[pl.ds(rbase + c * _LANES, _LANES)]
      tv = ti_v[pl.ds(rbase + c * _LANES, _LANES)]
      hb = lax.shift_left(lax.shift_right_logical(hv, 3), 3)
      tb = lax.shift_left(lax.shift_right_logical(tv, 3), 3)
      for j in range(_LANES):
        e = c * _LANES + j
        cps.append(pltpu.async_copy(
            ent.at[pl.ds(pl.multiple_of(hb[j], 8), 8), :],
            h_v.at[pl.ds(e * 8, 8), :], sem))
        cps.append(pltpu.async_copy(
            ent.at[pl.ds(pl.multiple_of(tb[j], 8), 8), :],
            t_v.at[pl.ds(e * 8, 8), :], sem))
    for cp in cps:
      cp.wait()
    c3.wait()
    c4.wait()

    def chunk(c, carry2):
      s = rbase + c * _LANES
      hrow = hi_v[pl.ds(s, _LANES)] & 7
      trow = ti_v[pl.ds(s, _LANES)] & 7
      orr = (ri_v[pl.ds(s, _LANES)] & 3) << 5
      om = (mi_v[pl.ds(s, _LANES)] & 3) << 5
      for j in range(_LANES):
        e = c * _LANES + j
        hr = e * 8 + hrow[j]
        tr = e * 8 + trow[j]
        offr = orr[j]
        offm = om[j]
        h0 = h_v[hr, pl.ds(0, _LANES)]
        h1 = h_v[hr, pl.ds(_LANES, _LANES)]
        t0 = t_v[tr, pl.ds(0, _LANES)]
        t1 = t_v[tr, pl.ds(_LANES, _LANES)]
        r0 = r_v[e, pl.ds(offr, _LANES)]
        r1 = r_v[e, pl.ds(offr + _LANES, _LANES)]
        m0 = m_v[e, pl.ds(offm, _LANES)]
        m1 = m_v[e, pl.ds(offm + _LANES, _LANES)]
        sv = (0.0 - jnp.abs(h0 + r0 + m0 - t0)) - jnp.abs(h1 + r1 + m1 - t1)
        cs = plsc.cumsum(sv)
        plsc.store_scatter(
            o_v, [jnp.full((_LANES,), rbase + e, jnp.int32)], cs,
            mask=last_lane)
      return carry2

    lax.fori_loop(0, _RCHUNKS, chunk, 0)
    return carry

  lax.fori_loop(0, _NROUNDS, do_round, 0)
  pltpu.sync_copy(o_v, out.at[pl.ds(base, _BPW)])


_mesh = plsc.VectorSubcoreMesh(
    core_axis_name="c", subcore_axis_name="s",
    num_cores=_NUM_CORES, num_subcores=_NUM_SUBCORES)

_sc_call = pl.kernel(
    _tec_body,
    out_type=jax.ShapeDtypeStruct((_BATCH,), jnp.float32),
    mesh=_mesh,
    compiler_params=pltpu.CompilerParams(needs_layout_passes=False),
    scratch_types=[
        pltpu.VMEM((_BPW,), jnp.int32),
        pltpu.VMEM((_BPW,), jnp.int32),
        pltpu.VMEM((_BPW,), jnp.int32),
        pltpu.VMEM((_BPW,), jnp.int32),
        pltpu.VMEM((_ROUND,), jnp.int32),
        pltpu.VMEM((_ROUND,), jnp.int32),
        pltpu.VMEM((_ROUND * 8, _EMB), jnp.float32),
        pltpu.VMEM((_ROUND * 8, _EMB), jnp.float32),
        pltpu.VMEM((_ROUND, _WIDE), jnp.float32),
        pltpu.VMEM((_ROUND, _WIDE), jnp.float32),
        pltpu.VMEM((_BPW,), jnp.float32),
        pltpu.SemaphoreType.DMA,
    ],
)


@jax.jit
def kernel(h_idx, r_idx, t_idx, time_idx, entity_emb, relation_emb, time_emb):
  rel128 = relation_emb.reshape(250, _WIDE)
  tim128 = jnp.pad(time_emb.reshape(-1), (0, 96)).reshape(92, _WIDE)
  return _sc_call(
      h_idx.astype(jnp.int32), r_idx.astype(jnp.int32),
      t_idx.astype(jnp.int32), time_idx.astype(jnp.int32),
      entity_emb, rel128, tim128)
